# baseline (speedup 1.0000x reference)
import functools

import jax
import jax.numpy as jnp
from jax import lax
from jax.experimental import pallas as pl
from jax.experimental.pallas import tpu as pltpu
from jax.experimental.pallas import tpu_sc as plsc

N = 10000
D = 128
E = 320000
BN_EPS = 1e-5

NC = 2
NS = 16
NW = NC * NS
EPW = E // NW
W = 128
NWIN = 80
EPWP = NWIN * W
KS = 8
NSS = NWIN // KS
NP = 10240
RPS = NP // NS
ZB = RPS // W
L = 16


def _sc_aggregate(h, srcw, dstw):
  mesh = plsc.VectorSubcoreMesh(core_axis_name="c", subcore_axis_name="s")

  @functools.partial(
      pl.kernel,
      out_type=(
          jax.ShapeDtypeStruct((NC * NP, D), jnp.float32),
          jax.ShapeDtypeStruct((NC * NP, D), jnp.float32),
      ),
      mesh=mesh,
      scratch_types=[
          pltpu.VMEM((KS, W), jnp.int32),
          pltpu.VMEM((KS, W), jnp.int32),
          pltpu.VMEM((W, D), jnp.float32),
          pltpu.VMEM((W, D), jnp.float32),
          pltpu.VMEM_SHARED((NP, D), jnp.float32),
          pltpu.SemaphoreType.DMA,
          pltpu.SemaphoreType.DMA,
          pltpu.SemaphoreType.DMA,
          pltpu.SemaphoreType.DMA,
      ],
  )
  def agg_kernel(h_hbm, srcw_hbm, dstw_hbm, p_hbm, c_hbm,
                 srcs_v, dsts_v, rows0, rows1, acc_sh,
                 gsem0, gsem1, ssem0, ssem1):
    cid = lax.axis_index("c")
    sid = lax.axis_index("s")
    wid = cid * NS + sid
    wbase = wid * NWIN
    row0 = sid * RPS
    obase = cid * NP + row0
    rows = (rows0, rows1)
    gsem = (gsem0, gsem1)
    ssem = (ssem0, ssem1)

    def fill(ref, val16):
      @pl.loop(0, W)
      def _(r):
        for cc in range(D // L):
          ref[r, pl.ds(cc * L, L)] = val16

    def zero_my_slice():
      for t in range(ZB):
        pltpu.sync_copy(rows1, acc_sh.at[pl.ds(row0 + t * W, W)])

    z16 = jnp.zeros((L,), jnp.float32)
    o16 = jnp.ones((L,), jnp.float32)

    fill(rows1, z16)
    zero_my_slice()
    plsc.subcore_barrier()

    @pl.loop(0, NSS)
    def _(s):
      swb = wbase + s * KS
      pltpu.sync_copy(srcw_hbm.at[pl.ds(swb, KS)], srcs_v)
      pltpu.sync_copy(dstw_hbm.at[pl.ds(swb, KS)], dsts_v)
      gh = [None, None]
      sh = [None, None]
      for j in range(KS):
        b = j & 1
        if sh[b] is not None:
          sh[b].wait()
        gh[b] = pltpu.async_copy(h_hbm.at[srcs_v.at[j]], rows[b], gsem[b])
        if j >= 1:
          pb = 1 - b
          gh[pb].wait()
          sh[pb] = pltpu.async_copy(rows[pb], acc_sh.at[dsts_v.at[j - 1]],
                                    ssem[pb], add=True)
      gh[1].wait()
      sh[1] = pltpu.async_copy(rows[1], acc_sh.at[dsts_v.at[KS - 1]],
                               ssem[1], add=True)
      sh[0].wait()
      sh[1].wait()

    plsc.subcore_barrier()
    pltpu.sync_copy(acc_sh.at[pl.ds(row0, RPS)], p_hbm.at[pl.ds(obase, RPS)])

    fill(rows1, z16)
    zero_my_slice()
    fill(rows0, o16)
    plsc.subcore_barrier()

    @pl.loop(0, NSS)
    def _(s):
      swb = wbase + s * KS
      pltpu.sync_copy(dstw_hbm.at[pl.ds(swb, KS)], dsts_v)
      sh = [None, None]
      for j in range(KS):
        b = j & 1
        if sh[b] is not None:
          sh[b].wait()
        sh[b] = pltpu.async_copy(rows0, acc_sh.at[dsts_v.at[j]],
                                 ssem[b], add=True)
      sh[0].wait()
      sh[1].wait()

    plsc.subcore_barrier()
    pltpu.sync_copy(acc_sh.at[pl.ds(row0, RPS)], c_hbm.at[pl.ds(obase, RPS)])

  p, c = agg_kernel(h, srcw, dstw)
  return p.reshape(NC, NP, D), c.reshape(NC, NP, D)


def _tc_body(h_ref, p_ref, c_ref, wl_ref, bl_ref, wr_ref, ga_ref, be_ref,
             mu_ref, va_ref, o_ref):
  cnt = jnp.maximum(c_ref[0, :, 0:1] + c_ref[1, :, 0:1], 1.0)
  agg = (p_ref[0] + p_ref[1]) / cnt
  hb = h_ref[...]
  dims = (((1,), (1,)), ((), ()))
  out = (lax.dot_general(agg, wl_ref[...], dims,
                         preferred_element_type=jnp.float32)
         + bl_ref[...]
         + lax.dot_general(hb, wr_ref[...], dims,
                           preferred_element_type=jnp.float32))
  s = ga_ref[...] * lax.rsqrt(va_ref[...] + BN_EPS)
  t = be_ref[...] - mu_ref[...] * s
  o_ref[...] = jnp.maximum(out * s + t, 0.0) + hb


def _tc_combine(h, p, c, W_l, b_l, W_r, gamma, beta, mu, var):
  BR = 1024
  full = lambda i: (0, 0)
  return pl.pallas_call(
      _tc_body,
      grid=(NP // BR,),
      in_specs=[
          pl.BlockSpec((BR, D), lambda i: (i, 0)),
          pl.BlockSpec((NC, BR, D), lambda i: (0, i, 0)),
          pl.BlockSpec((NC, BR, D), lambda i: (0, i, 0)),
          pl.BlockSpec((D, D), full),
          pl.BlockSpec((1, D), full),
          pl.BlockSpec((D, D), full),
          pl.BlockSpec((1, D), full),
          pl.BlockSpec((1, D), full),
          pl.BlockSpec((1, D), full),
          pl.BlockSpec((1, D), full),
      ],
      out_specs=pl.BlockSpec((BR, D), lambda i: (i, 0)),
      out_shape=jax.ShapeDtypeStruct((N, D), jnp.float32),
  )(h, p, c, W_l, b_l.reshape(1, D), W_r, gamma.reshape(1, D),
    beta.reshape(1, D), mu.reshape(1, D), var.reshape(1, D))


def kernel(h, edge_index, W_l, b_l, W_r, gamma, beta, running_mean,
           running_var):
  src = edge_index[0].reshape(NW, EPW)
  dst = edge_index[1].reshape(NW, EPW)
  pad = EPWP - EPW
  srcw = jnp.pad(src, ((0, 0), (0, pad))).reshape(NW * NWIN, W)
  dstw = jnp.pad(dst, ((0, 0), (0, pad)),
                 constant_values=NP - 1).reshape(NW * NWIN, W)
  p, c = _sc_aggregate(h, srcw, dstw)
  return _tc_combine(h, p, c, W_l, b_l, W_r, gamma, beta, running_mean,
                     running_var)


# speedup vs baseline: 1.2446x; 1.2446x over previous
import functools

import jax
import jax.numpy as jnp
from jax import lax
from jax.experimental import pallas as pl
from jax.experimental.pallas import tpu as pltpu
from jax.experimental.pallas import tpu_sc as plsc

N = 10000
D = 128
E = 320000
BN_EPS = 1e-5

NC = 2
NS = 16
NW = NC * NS
EPW = E // NW
W1 = 80
F1 = EPW // W1
W2 = 128
NWIN = 80
KS = 8
NSS = NWIN // KS
NP = 10240
RPS = NP // NS
L = 16


def _sc_aggregate(h, src, dst, dstw):
  mesh = plsc.VectorSubcoreMesh(core_axis_name="c", subcore_axis_name="s")

  @functools.partial(
      pl.kernel,
      out_type=(
          jax.ShapeDtypeStruct((NC * NP, D), jnp.float32),
          jax.ShapeDtypeStruct((NC * NP, D), jnp.float32),
      ),
      mesh=mesh,
      scratch_types=[
          pltpu.VMEM((W1,), jnp.int32),
          pltpu.VMEM((W1,), jnp.int32),
          pltpu.VMEM((W1, D), jnp.float32),
          pltpu.VMEM((KS, W2), jnp.int32),
          pltpu.VMEM((W2, D), jnp.float32),
          pltpu.VMEM_SHARED((NP, D), jnp.float32),
          pltpu.SemaphoreType.DMA,
          pltpu.SemaphoreType.DMA,
          pltpu.SemaphoreType.DMA,
      ],
  )
  def agg_kernel(h_hbm, src_hbm, dst_hbm, dstw_hbm, p_hbm, c_hbm,
                 src_v, dst_v, rows_v, dsts_v, ones_v, acc_sh,
                 gsem, ssem0, ssem1):
    cid = lax.axis_index("c")
    sid = lax.axis_index("s")
    wid = cid * NS + sid
    ebase = wid * EPW
    wbase = wid * NWIN
    row0 = sid * RPS
    obase = cid * NP + row0
    ssem = (ssem0, ssem1)

    def fill(ref, nrows, val16):
      @pl.loop(0, nrows)
      def _(r):
        for cc in range(D // L):
          ref[r, pl.ds(cc * L, L)] = val16

    def zero_my_slice():
      for t in range(RPS // W1):
        pltpu.sync_copy(rows_v, acc_sh.at[pl.ds(row0 + t * W1, W1)])

    z16 = jnp.zeros((L,), jnp.float32)
    o16 = jnp.ones((L,), jnp.float32)

    fill(rows_v, W1, z16)
    zero_my_slice()
    plsc.subcore_barrier()

    @pl.loop(0, F1)
    def _(i):
      b = ebase + i * W1
      pltpu.sync_copy(src_hbm.at[pl.ds(b, W1)], src_v)
      pltpu.sync_copy(dst_hbm.at[pl.ds(b, W1)], dst_v)
      pltpu.async_copy(h_hbm.at[src_v], rows_v, gsem).wait()
      pltpu.sync_copy(rows_v, acc_sh.at[dst_v], add=True)

    plsc.subcore_barrier()
    pltpu.sync_copy(acc_sh.at[pl.ds(row0, RPS)], p_hbm.at[pl.ds(obase, RPS)])

    fill(rows_v, W1, z16)
    zero_my_slice()
    fill(ones_v, W2, o16)
    plsc.subcore_barrier()

    @pl.loop(0, NSS)
    def _(s):
      swb = wbase + s * KS
      pltpu.sync_copy(dstw_hbm.at[pl.ds(swb, KS)], dsts_v)
      sh = [None, None]
      for j in range(KS):
        b = j & 1
        if sh[b] is not None:
          sh[b].wait()
        sh[b] = pltpu.async_copy(ones_v, acc_sh.at[dsts_v.at[j]],
                                 ssem[b], add=True)
      sh[0].wait()
      sh[1].wait()

    plsc.subcore_barrier()
    pltpu.sync_copy(acc_sh.at[pl.ds(row0, RPS)], c_hbm.at[pl.ds(obase, RPS)])

  p, c = agg_kernel(h, src, dst, dstw)
  return p.reshape(NC, NP, D), c.reshape(NC, NP, D)


def _tc_body(h_ref, p_ref, c_ref, wl_ref, bl_ref, wr_ref, ga_ref, be_ref,
             mu_ref, va_ref, o_ref):
  cnt = jnp.maximum(c_ref[0, :, 0:1] + c_ref[1, :, 0:1], 1.0)
  agg = (p_ref[0] + p_ref[1]) / cnt
  hb = h_ref[...]
  dims = (((1,), (1,)), ((), ()))
  out = (lax.dot_general(agg, wl_ref[...], dims,
                         preferred_element_type=jnp.float32)
         + bl_ref[...]
         + lax.dot_general(hb, wr_ref[...], dims,
                           preferred_element_type=jnp.float32))
  s = ga_ref[...] * lax.rsqrt(va_ref[...] + BN_EPS)
  t = be_ref[...] - mu_ref[...] * s
  o_ref[...] = jnp.maximum(out * s + t, 0.0) + hb


def _tc_combine(h, p, c, W_l, b_l, W_r, gamma, beta, mu, var):
  BR = 1024
  full = lambda i: (0, 0)
  return pl.pallas_call(
      _tc_body,
      grid=(NP // BR,),
      in_specs=[
          pl.BlockSpec((BR, D), lambda i: (i, 0)),
          pl.BlockSpec((NC, BR, D), lambda i: (0, i, 0)),
          pl.BlockSpec((NC, BR, D), lambda i: (0, i, 0)),
          pl.BlockSpec((D, D), full),
          pl.BlockSpec((1, D), full),
          pl.BlockSpec((D, D), full),
          pl.BlockSpec((1, D), full),
          pl.BlockSpec((1, D), full),
          pl.BlockSpec((1, D), full),
          pl.BlockSpec((1, D), full),
      ],
      out_specs=pl.BlockSpec((BR, D), lambda i: (i, 0)),
      out_shape=jax.ShapeDtypeStruct((N, D), jnp.float32),
  )(h, p, c, W_l, b_l.reshape(1, D), W_r, gamma.reshape(1, D),
    beta.reshape(1, D), mu.reshape(1, D), var.reshape(1, D))


def kernel(h, edge_index, W_l, b_l, W_r, gamma, beta, running_mean,
           running_var):
  src = edge_index[0]
  dst = edge_index[1]
  dstp = dst.reshape(NW, EPW)
  dstw = jnp.pad(dstp, ((0, 0), (0, NWIN * W2 - EPW)),
                 constant_values=NP - 1).reshape(NW * NWIN, W2)
  p, c = _sc_aggregate(h, src, dst, dstw)
  return _tc_combine(h, p, c, W_l, b_l, W_r, gamma, beta, running_mean,
                     running_var)


# speedup vs baseline: 1.9581x; 1.5733x over previous
import functools

import jax
import jax.numpy as jnp
from jax import lax
from jax.experimental import pallas as pl
from jax.experimental.pallas import tpu as pltpu
from jax.experimental.pallas import tpu_sc as plsc

N = 10000
D = 128
E = 320000
BN_EPS = 1e-5

NC = 2
NS = 16
NW = NC * NS
EPW = E // NW
W = 80
F = EPW // W
NB = 4
SS = F // NB
NP = 10240
RPS = NP // NS
L = 16


def _sc_aggregate(h, src, dst):
  mesh = plsc.VectorSubcoreMesh(core_axis_name="c", subcore_axis_name="s")

  @functools.partial(
      pl.kernel,
      out_type=(
          jax.ShapeDtypeStruct((NC * NP, D), jnp.float32),
          jax.ShapeDtypeStruct((NC * NP, D), jnp.float32),
      ),
      mesh=mesh,
      scratch_types=(
          [pltpu.VMEM((W,), jnp.int32) for _ in range(NB)]
          + [pltpu.VMEM((W,), jnp.int32) for _ in range(NB)]
          + [pltpu.VMEM((W, D), jnp.float32) for _ in range(NB)]
          + [pltpu.SemaphoreType.DMA for _ in range(3 * NB)]
          + [pltpu.VMEM_SHARED((NP, D), jnp.float32)]
      ),
  )
  def agg_kernel(h_hbm, src_hbm, dst_hbm, p_hbm, c_hbm, *bufs):
    srcv = bufs[0:NB]
    dstv = bufs[NB:2 * NB]
    rows = bufs[2 * NB:3 * NB]
    isem = bufs[3 * NB:4 * NB]
    gsem = bufs[4 * NB:5 * NB]
    ssem = bufs[5 * NB:6 * NB]
    acc_sh = bufs[6 * NB]

    cid = lax.axis_index("c")
    sid = lax.axis_index("s")
    wid = cid * NS + sid
    ebase = wid * EPW
    row0 = sid * RPS
    obase = cid * NP + row0

    def fill(ref, val16):
      @pl.loop(0, W)
      def _(r):
        for cc in range(D // L):
          ref[r, pl.ds(cc * L, L)] = val16

    def zero_my_slice():
      for t in range(RPS // W):
        pltpu.sync_copy(rows[1], acc_sh.at[pl.ds(row0 + t * W, W)])

    z16 = jnp.zeros((L,), jnp.float32)
    o16 = jnp.ones((L,), jnp.float32)

    fill(rows[1], z16)
    zero_my_slice()
    plsc.subcore_barrier()

    @pl.loop(0, SS)
    def _(s):
      base = ebase + s * NB * W
      ih = []
      for j in range(NB):
        bj = base + j * W
        h1 = pltpu.async_copy(src_hbm.at[pl.ds(bj, W)], srcv[j], isem[j])
        h2 = pltpu.async_copy(dst_hbm.at[pl.ds(bj, W)], dstv[j], isem[j])
        ih.append((h1, h2))
      gh = []
      for j in range(NB):
        ih[j][0].wait()
        ih[j][1].wait()
        gh.append(pltpu.async_copy(h_hbm.at[srcv[j]], rows[j], gsem[j]))
      sh = []
      for j in range(NB):
        gh[j].wait()
        sh.append(pltpu.async_copy(rows[j], acc_sh.at[dstv[j]], ssem[j],
                                   add=True))
      for j in range(NB):
        sh[j].wait()

    bt = ebase + SS * NB * W
    pltpu.sync_copy(src_hbm.at[pl.ds(bt, W)], srcv[0])
    pltpu.sync_copy(dst_hbm.at[pl.ds(bt, W)], dstv[0])
    pltpu.async_copy(h_hbm.at[srcv[0]], rows[0], gsem[0]).wait()
    pltpu.sync_copy(rows[0], acc_sh.at[dstv[0]], add=True)

    plsc.subcore_barrier()
    pltpu.sync_copy(acc_sh.at[pl.ds(row0, RPS)], p_hbm.at[pl.ds(obase, RPS)])

    fill(rows[1], z16)
    zero_my_slice()
    fill(rows[0], o16)
    plsc.subcore_barrier()

    @pl.loop(0, SS)
    def _(s):
      base = ebase + s * NB * W
      ih = []
      for j in range(NB):
        bj = base + j * W
        ih.append(pltpu.async_copy(dst_hbm.at[pl.ds(bj, W)], dstv[j],
                                   isem[j]))
      sh = []
      for j in range(NB):
        ih[j].wait()
        sh.append(pltpu.async_copy(rows[0], acc_sh.at[dstv[j]], ssem[j],
                                   add=True))
      for j in range(NB):
        sh[j].wait()

    pltpu.sync_copy(dst_hbm.at[pl.ds(bt, W)], dstv[0])
    pltpu.sync_copy(rows[0], acc_sh.at[dstv[0]], add=True)

    plsc.subcore_barrier()
    pltpu.sync_copy(acc_sh.at[pl.ds(row0, RPS)], c_hbm.at[pl.ds(obase, RPS)])

  p, c = agg_kernel(h, src, dst)
  return p.reshape(NC, NP, D), c.reshape(NC, NP, D)


def _tc_body(h_ref, p_ref, c_ref, wl_ref, bl_ref, wr_ref, ga_ref, be_ref,
             mu_ref, va_ref, o_ref):
  cnt = jnp.maximum(c_ref[0, :, 0:1] + c_ref[1, :, 0:1], 1.0)
  agg = (p_ref[0] + p_ref[1]) / cnt
  hb = h_ref[...]
  dims = (((1,), (1,)), ((), ()))
  out = (lax.dot_general(agg, wl_ref[...], dims,
                         preferred_element_type=jnp.float32)
         + bl_ref[...]
         + lax.dot_general(hb, wr_ref[...], dims,
                           preferred_element_type=jnp.float32))
  s = ga_ref[...] * lax.rsqrt(va_ref[...] + BN_EPS)
  t = be_ref[...] - mu_ref[...] * s
  o_ref[...] = jnp.maximum(out * s + t, 0.0) + hb


def _tc_combine(h, p, c, W_l, b_l, W_r, gamma, beta, mu, var):
  BR = 1024
  full = lambda i: (0, 0)
  return pl.pallas_call(
      _tc_body,
      grid=(NP // BR,),
      in_specs=[
          pl.BlockSpec((BR, D), lambda i: (i, 0)),
          pl.BlockSpec((NC, BR, D), lambda i: (0, i, 0)),
          pl.BlockSpec((NC, BR, D), lambda i: (0, i, 0)),
          pl.BlockSpec((D, D), full),
          pl.BlockSpec((1, D), full),
          pl.BlockSpec((D, D), full),
          pl.BlockSpec((1, D), full),
          pl.BlockSpec((1, D), full),
          pl.BlockSpec((1, D), full),
          pl.BlockSpec((1, D), full),
      ],
      out_specs=pl.BlockSpec((BR, D), lambda i: (i, 0)),
      out_shape=jax.ShapeDtypeStruct((N, D), jnp.float32),
  )(h, p, c, W_l, b_l.reshape(1, D), W_r, gamma.reshape(1, D),
    beta.reshape(1, D), mu.reshape(1, D), var.reshape(1, D))


def kernel(h, edge_index, W_l, b_l, W_r, gamma, beta, running_mean,
           running_var):
  src = edge_index[0]
  dst = edge_index[1]
  p, c = _sc_aggregate(h, src, dst)
  return _tc_combine(h, p, c, W_l, b_l, W_r, gamma, beta, running_mean,
                     running_var)
